# baseline (speedup 1.0000x reference)
import functools

import jax
import jax.numpy as jnp
from jax import lax
from jax.experimental import pallas as pl
from jax.experimental.pallas import tpu as pltpu
from jax.experimental.pallas import tpu_sc as plsc

B = 16384
F = 26
V = 100000
D = 32
OUT = 64
EPS = 1e-5

NC = 2
NS = 16
NW = NC * NS
HB = B // 2


def _sc_gather_t(idx_flat, tabT2):
    mesh = plsc.VectorSubcoreMesh(core_axis_name="c", subcore_axis_name="s")

    @functools.partial(
        pl.kernel,
        out_type=jax.ShapeDtypeStruct((F * D * B,), jnp.float32),
        mesh=mesh,
        scratch_types=[
            pltpu.VMEM((1, V), jnp.float32),
            pltpu.VMEM((HB,), jnp.int32),
            pltpu.VMEM((HB,), jnp.float32),
            pltpu.VMEM((HB,), jnp.float32),
            pltpu.SemaphoreType.DMA,
            pltpu.SemaphoreType.DMA,
            pltpu.SemaphoreType.DMA,
        ],
        compiler_params=pltpu.CompilerParams(use_tc_tiling_on_sc=True,
                                             needs_layout_passes=False),
    )
    def k(idx_hbm, tab_hbm, out_hbm, row_v, idx_v, out0, out1, os0, os1,
          rs):
        d = lax.axis_index("s") * NC + lax.axis_index("c")
        outs = (out0, out1)
        osems = (os0, os1)
        wc = [None, None]
        QC = 25088
        bounds = [(q * QC, min((q + 1) * QC, V)) for q in range(4)]
        for i in range(F):
            r = i * D + d
            rcs = [pltpu.async_copy(
                tab_hbm.at[pl.ds(r, 1), pl.ds(lo, hi - lo)],
                row_v.at[:, pl.ds(lo, hi - lo)], rs) for lo, hi in bounds]
            for c in rcs:
                c.wait()
            for h in range(2):
                pltpu.sync_copy(idx_hbm.at[pl.ds(i * B + h * HB, HB)], idx_v)
                if wc[h] is not None:
                    wc[h].wait()
                out_v = outs[h]
                zz = jnp.zeros((16,), jnp.int32)

                def body(j, carry):
                    base = j * 128
                    for u in range(8):
                        o = base + u * 16
                        iv = idx_v[pl.ds(o, 16)]
                        out_v[pl.ds(o, 16)] = plsc.load_gather(row_v, [zz, iv])
                    return carry

                lax.fori_loop(0, HB // 128, body, 0)
                wc[h] = pltpu.async_copy(
                    out_v, out_hbm.at[pl.ds(r * B + h * HB, HB)], osems[h])
        wc[0].wait()
        wc[1].wait()

    return k(idx_flat, tabT2)


XB = 2048


def _tc_head_body(x_ref, w_ref, b_ref, ga_ref, be_ref, rm_ref, rv_ref, o_ref):
    acc = lax.dot_general(x_ref[...], w_ref[...],
                          dimension_numbers=(((0,), (1,)), ((), ())),
                          preferred_element_type=jnp.float32)
    h = jnp.maximum(acc + b_ref[0], 0.0)
    scale = ga_ref[0] * lax.rsqrt(rv_ref[0] + EPS)
    shift = be_ref[0] - rm_ref[0] * scale
    o_ref[...] = h * scale + shift


def _tc_head(xT, W, b, gamma, beta, rm, rv):
    return pl.pallas_call(
        _tc_head_body,
        grid=(B // XB,),
        in_specs=[
            pl.BlockSpec((F * D, XB), lambda i: (0, i)),
            pl.BlockSpec((OUT, F * D), lambda i: (0, 0)),
            pl.BlockSpec((1, OUT), lambda i: (0, 0)),
            pl.BlockSpec((1, OUT), lambda i: (0, 0)),
            pl.BlockSpec((1, OUT), lambda i: (0, 0)),
            pl.BlockSpec((1, OUT), lambda i: (0, 0)),
            pl.BlockSpec((1, OUT), lambda i: (0, 0)),
        ],
        out_specs=pl.BlockSpec((XB, OUT), lambda i: (i, 0)),
        out_shape=jax.ShapeDtypeStruct((B, OUT), jnp.float32),
    )(xT, W, b, gamma, beta, rm, rv)


def kernel(cat0, cat1, cat2, cat3, cat4, cat5, cat6, cat7, cat8, cat9,
           cat10, cat11, cat12, cat13, cat14, cat15, cat16, cat17, cat18,
           cat19, cat20, cat21, cat22, cat23, cat24, cat25,
           tables, W, b, gamma, beta, running_mean, running_var):
    cats = jnp.stack(
        [cat0, cat1, cat2, cat3, cat4, cat5, cat6, cat7, cat8, cat9,
         cat10, cat11, cat12, cat13, cat14, cat15, cat16, cat17, cat18,
         cat19, cat20, cat21, cat22, cat23, cat24, cat25], axis=0)
    idx_flat = cats.reshape(F * B)
    tabT2 = tables.transpose(0, 2, 1).reshape(F * D, V)
    g = _sc_gather_t(idx_flat, tabT2)
    xT = g.reshape(F * D, B)
    return _tc_head(xT, W, b[None], gamma[None], beta[None],
                    running_mean[None], running_var[None])


# speedup vs baseline: 1.2109x; 1.2109x over previous
import functools

import jax
import jax.numpy as jnp
from jax import lax
from jax.experimental import pallas as pl
from jax.experimental.pallas import tpu as pltpu
from jax.experimental.pallas import tpu_sc as plsc

B = 16384
F = 26
V = 100000
D = 32
OUT = 64
EPS = 1e-5

NC = 2
NS = 16
NW = NC * NS
HB = B // 2


def _sc_gather_t(idx_flat, tabT2):
    mesh = plsc.VectorSubcoreMesh(core_axis_name="c", subcore_axis_name="s")

    @functools.partial(
        pl.kernel,
        out_type=jax.ShapeDtypeStruct((F * D * B,), jnp.float32),
        mesh=mesh,
        scratch_types=[
            pltpu.VMEM((1, V), jnp.float32),
            pltpu.VMEM((HB,), jnp.int32),
            pltpu.VMEM((HB,), jnp.float32),
            pltpu.VMEM((HB,), jnp.float32),
            pltpu.SemaphoreType.DMA,
            pltpu.SemaphoreType.DMA,
            pltpu.SemaphoreType.DMA,
        ],
        compiler_params=pltpu.CompilerParams(use_tc_tiling_on_sc=True,
                                             needs_layout_passes=False),
    )
    def k(idx_hbm, tab_hbm, out_hbm, row_v, idx_v, out0, out1, os0, os1,
          rs):
        d = lax.axis_index("s") * NC + lax.axis_index("c")
        outs = (out0, out1)
        osems = (os0, os1)
        wc = [None, None]
        QC = 25088
        bounds = [(q * QC, min((q + 1) * QC, V)) for q in range(4)]
        for i in range(F):
            r = i * D + d
            rcs = [pltpu.async_copy(
                tab_hbm.at[pl.ds(r, 1), pl.ds(lo, hi - lo)],
                row_v.at[:, pl.ds(lo, hi - lo)], rs) for lo, hi in bounds]
            for c in rcs:
                c.wait()
            for h in range(2):
                pltpu.sync_copy(idx_hbm.at[pl.ds(i * B + h * HB, HB)], idx_v)
                if wc[h] is not None:
                    wc[h].wait()
                out_v = outs[h]
                zz = jnp.zeros((16,), jnp.int32)

                @plsc.parallel_loop(0, HB, step=16, unroll=8)
                def _(o):
                    iv = idx_v[pl.ds(o, 16)]
                    out_v[pl.ds(o, 16)] = plsc.load_gather(row_v, [zz, iv])
                wc[h] = pltpu.async_copy(
                    out_v, out_hbm.at[pl.ds(r * B + h * HB, HB)], osems[h])
        wc[0].wait()
        wc[1].wait()

    return k(idx_flat, tabT2)


XB = 2048


def _tc_head_body(x_ref, w_ref, b_ref, ga_ref, be_ref, rm_ref, rv_ref, o_ref):
    acc = lax.dot_general(x_ref[...], w_ref[...],
                          dimension_numbers=(((0,), (1,)), ((), ())),
                          preferred_element_type=jnp.float32)
    h = jnp.maximum(acc + b_ref[0], 0.0)
    scale = ga_ref[0] * lax.rsqrt(rv_ref[0] + EPS)
    shift = be_ref[0] - rm_ref[0] * scale
    o_ref[...] = h * scale + shift


def _tc_head(xT, W, b, gamma, beta, rm, rv):
    return pl.pallas_call(
        _tc_head_body,
        grid=(B // XB,),
        in_specs=[
            pl.BlockSpec((F * D, XB), lambda i: (0, i)),
            pl.BlockSpec((OUT, F * D), lambda i: (0, 0)),
            pl.BlockSpec((1, OUT), lambda i: (0, 0)),
            pl.BlockSpec((1, OUT), lambda i: (0, 0)),
            pl.BlockSpec((1, OUT), lambda i: (0, 0)),
            pl.BlockSpec((1, OUT), lambda i: (0, 0)),
            pl.BlockSpec((1, OUT), lambda i: (0, 0)),
        ],
        out_specs=pl.BlockSpec((XB, OUT), lambda i: (i, 0)),
        out_shape=jax.ShapeDtypeStruct((B, OUT), jnp.float32),
    )(xT, W, b, gamma, beta, rm, rv)


def kernel(cat0, cat1, cat2, cat3, cat4, cat5, cat6, cat7, cat8, cat9,
           cat10, cat11, cat12, cat13, cat14, cat15, cat16, cat17, cat18,
           cat19, cat20, cat21, cat22, cat23, cat24, cat25,
           tables, W, b, gamma, beta, running_mean, running_var):
    cats = jnp.stack(
        [cat0, cat1, cat2, cat3, cat4, cat5, cat6, cat7, cat8, cat9,
         cat10, cat11, cat12, cat13, cat14, cat15, cat16, cat17, cat18,
         cat19, cat20, cat21, cat22, cat23, cat24, cat25], axis=0)
    idx_flat = cats.reshape(F * B)
    tabT2 = tables.transpose(0, 2, 1).reshape(F * D, V)
    g = _sc_gather_t(idx_flat, tabT2)
    xT = g.reshape(F * D, B)
    return _tc_head(xT, W, b[None], gamma[None], beta[None],
                    running_mean[None], running_var[None])


# speedup vs baseline: 1.2290x; 1.0149x over previous
import functools

import jax
import jax.numpy as jnp
from jax import lax
from jax.experimental import pallas as pl
from jax.experimental.pallas import tpu as pltpu
from jax.experimental.pallas import tpu_sc as plsc

B = 16384
F = 26
V = 100000
D = 32
OUT = 64
EPS = 1e-5

NC = 2
NS = 16
NW = NC * NS
HB = B // 2
NF = 13


def _sc_gather_t(cats, tabT2, fo):
    mesh = plsc.VectorSubcoreMesh(core_axis_name="c", subcore_axis_name="s")

    @functools.partial(
        pl.kernel,
        out_type=jax.ShapeDtypeStruct((NF * D * B,), jnp.float32),
        mesh=mesh,
        scratch_types=[
            pltpu.VMEM((1, V), jnp.float32),
            pltpu.VMEM((HB,), jnp.int32),
            pltpu.VMEM((HB,), jnp.float32),
            pltpu.VMEM((HB,), jnp.float32),
            pltpu.SemaphoreType.DMA,
            pltpu.SemaphoreType.DMA,
            pltpu.SemaphoreType.DMA,
        ],
        compiler_params=pltpu.CompilerParams(use_tc_tiling_on_sc=True,
                                             needs_layout_passes=False),
    )
    def k(*refs):
        idx_hbms = refs[:NF]
        tab_hbm = refs[NF]
        out_hbm = refs[NF + 1]
        row_v, idx_v, out0, out1, os0, os1, rs = refs[NF + 2:]
        d = lax.axis_index("s") * NC + lax.axis_index("c")
        outs = (out0, out1)
        osems = (os0, os1)
        wc = [None, None]
        QC = 25088
        bounds = [(q * QC, min((q + 1) * QC, V)) for q in range(4)]
        for i in range(NF):
            r = (fo + i) * D + d
            rcs = [pltpu.async_copy(
                tab_hbm.at[pl.ds(r, 1), pl.ds(lo, hi - lo)],
                row_v.at[:, pl.ds(lo, hi - lo)], rs) for lo, hi in bounds]
            for c in rcs:
                c.wait()
            rloc = i * D + d
            for h in range(2):
                pltpu.sync_copy(idx_hbms[i].at[pl.ds(h * HB, HB)], idx_v)
                if wc[h] is not None:
                    wc[h].wait()
                out_v = outs[h]
                zz = jnp.zeros((16,), jnp.int32)

                @plsc.parallel_loop(0, HB, step=16, unroll=8)
                def _(o):
                    iv = idx_v[pl.ds(o, 16)]
                    out_v[pl.ds(o, 16)] = plsc.load_gather(row_v, [zz, iv])
                wc[h] = pltpu.async_copy(
                    out_v, out_hbm.at[pl.ds(rloc * B + h * HB, HB)], osems[h])
        wc[0].wait()
        wc[1].wait()

    return k(*cats, tabT2)


XB = 2048


def _tc_partial_body(x_ref, w_ref, o_ref):
    o_ref[...] = lax.dot_general(
        x_ref[...], w_ref[...],
        dimension_numbers=(((0,), (1,)), ((), ())),
        preferred_element_type=jnp.float32)


def _tc_partial(xT, Wh):
    return pl.pallas_call(
        _tc_partial_body,
        grid=(B // XB,),
        in_specs=[
            pl.BlockSpec((NF * D, XB), lambda i: (0, i)),
            pl.BlockSpec((OUT, NF * D), lambda i: (0, 0)),
        ],
        out_specs=pl.BlockSpec((XB, OUT), lambda i: (i, 0)),
        out_shape=jax.ShapeDtypeStruct((B, OUT), jnp.float32),
    )(xT, Wh)


def _tc_final_body(x_ref, w_ref, p_ref, b_ref, ga_ref, be_ref, rm_ref,
                   rv_ref, o_ref):
    acc = p_ref[...] + lax.dot_general(
        x_ref[...], w_ref[...],
        dimension_numbers=(((0,), (1,)), ((), ())),
        preferred_element_type=jnp.float32)
    h = jnp.maximum(acc + b_ref[0], 0.0)
    scale = ga_ref[0] * lax.rsqrt(rv_ref[0] + EPS)
    shift = be_ref[0] - rm_ref[0] * scale
    o_ref[...] = h * scale + shift


def _tc_final(xT, Wh, part, b, gamma, beta, rm, rv):
    return pl.pallas_call(
        _tc_final_body,
        grid=(B // XB,),
        in_specs=[
            pl.BlockSpec((NF * D, XB), lambda i: (0, i)),
            pl.BlockSpec((OUT, NF * D), lambda i: (0, 0)),
            pl.BlockSpec((XB, OUT), lambda i: (i, 0)),
            pl.BlockSpec((1, OUT), lambda i: (0, 0)),
            pl.BlockSpec((1, OUT), lambda i: (0, 0)),
            pl.BlockSpec((1, OUT), lambda i: (0, 0)),
            pl.BlockSpec((1, OUT), lambda i: (0, 0)),
            pl.BlockSpec((1, OUT), lambda i: (0, 0)),
        ],
        out_specs=pl.BlockSpec((XB, OUT), lambda i: (i, 0)),
        out_shape=jax.ShapeDtypeStruct((B, OUT), jnp.float32),
    )(xT, Wh, part, b, gamma, beta, rm, rv)


def kernel(cat0, cat1, cat2, cat3, cat4, cat5, cat6, cat7, cat8, cat9,
           cat10, cat11, cat12, cat13, cat14, cat15, cat16, cat17, cat18,
           cat19, cat20, cat21, cat22, cat23, cat24, cat25,
           tables, W, b, gamma, beta, running_mean, running_var):
    cats = [cat0, cat1, cat2, cat3, cat4, cat5, cat6, cat7, cat8, cat9,
            cat10, cat11, cat12, cat13, cat14, cat15, cat16, cat17, cat18,
            cat19, cat20, cat21, cat22, cat23, cat24, cat25]
    tabT2 = tables.transpose(0, 2, 1).reshape(F * D, V)
    gA = _sc_gather_t(cats[:NF], tabT2, 0)
    gB = _sc_gather_t(cats[NF:], tabT2, NF)
    xA = gA.reshape(NF * D, B)
    xB = gB.reshape(NF * D, B)
    part = _tc_partial(xA, W[:, :NF * D])
    return _tc_final(xB, W[:, NF * D:], part, b[None], gamma[None],
                     beta[None], running_mean[None], running_var[None])


# speedup vs baseline: 1.2431x; 1.0115x over previous
import functools

import jax
import jax.numpy as jnp
from jax import lax
from jax.experimental import pallas as pl
from jax.experimental.pallas import tpu as pltpu
from jax.experimental.pallas import tpu_sc as plsc

B = 16384
F = 26
V = 100000
D = 32
OUT = 64
EPS = 1e-5

NC = 2
NS = 16
NW = NC * NS
HB = B // 2
NFA = 16
NFB = F - NFA


def _sc_gather_t(cats, tabT2, fo):
    nf = len(cats)
    mesh = plsc.VectorSubcoreMesh(core_axis_name="c", subcore_axis_name="s")

    @functools.partial(
        pl.kernel,
        out_type=jax.ShapeDtypeStruct((nf * D * B,), jnp.float32),
        mesh=mesh,
        scratch_types=[
            pltpu.VMEM((1, V), jnp.float32),
            pltpu.VMEM((HB,), jnp.int32),
            pltpu.VMEM((HB,), jnp.float32),
            pltpu.VMEM((HB,), jnp.float32),
            pltpu.SemaphoreType.DMA,
            pltpu.SemaphoreType.DMA,
            pltpu.SemaphoreType.DMA,
        ],
        compiler_params=pltpu.CompilerParams(use_tc_tiling_on_sc=True,
                                             needs_layout_passes=False),
    )
    def k(*refs):
        idx_hbms = refs[:nf]
        tab_hbm = refs[nf]
        out_hbm = refs[nf + 1]
        row_v, idx_v, out0, out1, os0, os1, rs = refs[nf + 2:]
        d = lax.axis_index("s") * NC + lax.axis_index("c")
        outs = (out0, out1)
        osems = (os0, os1)
        wc = [None, None]
        QC = 25088
        bounds = [(q * QC, min((q + 1) * QC, V)) for q in range(4)]
        for i in range(nf):
            r = (fo + i) * D + d
            rcs = [pltpu.async_copy(
                tab_hbm.at[pl.ds(r, 1), pl.ds(lo, hi - lo)],
                row_v.at[:, pl.ds(lo, hi - lo)], rs) for lo, hi in bounds]
            for c in rcs:
                c.wait()
            rloc = i * D + d
            for h in range(2):
                pltpu.sync_copy(idx_hbms[i].at[pl.ds(h * HB, HB)], idx_v)
                if wc[h] is not None:
                    wc[h].wait()
                out_v = outs[h]
                zz = jnp.zeros((16,), jnp.int32)

                @plsc.parallel_loop(0, HB, step=16, unroll=8)
                def _(o):
                    iv = idx_v[pl.ds(o, 16)]
                    out_v[pl.ds(o, 16)] = plsc.load_gather(row_v, [zz, iv])
                wc[h] = pltpu.async_copy(
                    out_v, out_hbm.at[pl.ds(rloc * B + h * HB, HB)], osems[h])
        wc[0].wait()
        wc[1].wait()

    return k(*cats, tabT2)


XB = 2048


def _tc_partial_body(x_ref, w_ref, o_ref):
    o_ref[...] = lax.dot_general(
        x_ref[...], w_ref[...],
        dimension_numbers=(((0,), (1,)), ((), ())),
        preferred_element_type=jnp.float32)


def _tc_partial(xT, Wh):
    nfd = xT.shape[0]
    return pl.pallas_call(
        _tc_partial_body,
        grid=(B // XB,),
        in_specs=[
            pl.BlockSpec((nfd, XB), lambda i: (0, i)),
            pl.BlockSpec((OUT, nfd), lambda i: (0, 0)),
        ],
        out_specs=pl.BlockSpec((XB, OUT), lambda i: (i, 0)),
        out_shape=jax.ShapeDtypeStruct((B, OUT), jnp.float32),
    )(xT, Wh)


def _tc_final_body(x_ref, w_ref, p_ref, b_ref, ga_ref, be_ref, rm_ref,
                   rv_ref, o_ref):
    acc = p_ref[...] + lax.dot_general(
        x_ref[...], w_ref[...],
        dimension_numbers=(((0,), (1,)), ((), ())),
        preferred_element_type=jnp.float32)
    h = jnp.maximum(acc + b_ref[0], 0.0)
    scale = ga_ref[0] * lax.rsqrt(rv_ref[0] + EPS)
    shift = be_ref[0] - rm_ref[0] * scale
    o_ref[...] = h * scale + shift


def _tc_final(xT, Wh, part, b, gamma, beta, rm, rv):
    nfd = xT.shape[0]
    return pl.pallas_call(
        _tc_final_body,
        grid=(B // XB,),
        in_specs=[
            pl.BlockSpec((nfd, XB), lambda i: (0, i)),
            pl.BlockSpec((OUT, nfd), lambda i: (0, 0)),
            pl.BlockSpec((XB, OUT), lambda i: (i, 0)),
            pl.BlockSpec((1, OUT), lambda i: (0, 0)),
            pl.BlockSpec((1, OUT), lambda i: (0, 0)),
            pl.BlockSpec((1, OUT), lambda i: (0, 0)),
            pl.BlockSpec((1, OUT), lambda i: (0, 0)),
            pl.BlockSpec((1, OUT), lambda i: (0, 0)),
        ],
        out_specs=pl.BlockSpec((XB, OUT), lambda i: (i, 0)),
        out_shape=jax.ShapeDtypeStruct((B, OUT), jnp.float32),
    )(xT, Wh, part, b, gamma, beta, rm, rv)


def kernel(cat0, cat1, cat2, cat3, cat4, cat5, cat6, cat7, cat8, cat9,
           cat10, cat11, cat12, cat13, cat14, cat15, cat16, cat17, cat18,
           cat19, cat20, cat21, cat22, cat23, cat24, cat25,
           tables, W, b, gamma, beta, running_mean, running_var):
    cats = [cat0, cat1, cat2, cat3, cat4, cat5, cat6, cat7, cat8, cat9,
            cat10, cat11, cat12, cat13, cat14, cat15, cat16, cat17, cat18,
            cat19, cat20, cat21, cat22, cat23, cat24, cat25]
    tabT2 = tables.transpose(0, 2, 1).reshape(F * D, V)
    gA = _sc_gather_t(cats[:NFA], tabT2, 0)
    gB = _sc_gather_t(cats[NFA:], tabT2, NFA)
    xA = gA.reshape(NFA * D, B)
    xB = gB.reshape(NFB * D, B)
    part = _tc_partial(xA, W[:, :NFA * D])
    return _tc_final(xB, W[:, NFA * D:], part, b[None], gamma[None],
                     beta[None], running_mean[None], running_var[None])
